# Initial kernel scaffold; baseline (speedup 1.0000x reference)
import jax
import jax.numpy as jnp
from jax.experimental import pallas as pl


def kernel(x, edge_index_0, edge_index_1, edge_index_2, W0_0, W0_1, W1_0, W1_1, W2_0, W2_1, att):
    raise NotImplementedError("write your pallas kernel here")


# speedup vs baseline: 5.1065x; 5.1065x over previous
import functools

import jax
import jax.numpy as jnp
from jax import lax
from jax.experimental import pallas as pl
from jax.experimental.pallas import tpu as pltpu
from jax.experimental.pallas import tpu_sc as plsc

N = 10000
NP = 10240
D = 128
E = 320000
C = 3

NC = 2
NS = 16
NW = NC * NS
EPW = E // NW
K = 80
NCHUNK = EPW // K
RPT = NP // NS
DEG_W = 16

_mesh = functools.partial(
    plsc.VectorSubcoreMesh, core_axis_name="c", subcore_axis_name="s",
    num_cores=NC, num_subcores=NS)


def _agg_body(t_hbm, src_hbm, dst_hbm, zero_hbm, out_hbm,
              src_v, dst_v, rows_v, acc_sp):
    c = lax.axis_index("c")
    s = lax.axis_index("s")
    wid = c * NS + s
    pltpu.sync_copy(src_hbm.at[wid], src_v)
    pltpu.sync_copy(dst_hbm.at[wid], dst_v)
    pltpu.sync_copy(zero_hbm.at[pl.ds(s * RPT, RPT)],
                    acc_sp.at[pl.ds(s * RPT, RPT)])
    plsc.subcore_barrier()

    @pl.loop(0, NCHUNK)
    def _chunk(i):
        pltpu.sync_copy(t_hbm.at[src_v.at[i]], rows_v)
        pltpu.sync_copy(rows_v, acc_sp.at[dst_v.at[i]], add=True)

    plsc.subcore_barrier()
    pltpu.sync_copy(acc_sp.at[pl.ds(s * RPT, RPT)],
                    out_hbm.at[c, pl.ds(s * RPT, RPT)])


@jax.jit
def _agg(t, src3, dst3, zero_agg):
    return pl.kernel(
        _agg_body,
        out_type=jax.ShapeDtypeStruct((NC, NP, D), jnp.float32),
        mesh=_mesh(),
        scratch_types=[
            pltpu.VMEM((NCHUNK, K), jnp.int32),
            pltpu.VMEM((NCHUNK, K), jnp.int32),
            pltpu.VMEM((K, D), jnp.float32),
            pltpu.VMEM_SHARED((NP, D), jnp.float32),
        ],
    )(t, src3, dst3, zero_agg)


def _deg_body(dst_hbm, out_hbm, dst_v, hist):
    c = lax.axis_index("c")
    s = lax.axis_index("s")
    wid = c * NS + s
    ones = jnp.ones((16,), jnp.float32)
    for ch in range(C):
        pltpu.sync_copy(dst_hbm.at[ch, wid], dst_v)

        @pl.loop(0, NP // 16)
        def _z(i):
            hist[pl.ds(i * 16, 16)] = jnp.zeros((16,), jnp.float32)

        @pl.loop(0, NCHUNK)
        def _acc(i):
            for j in range(K // 16):
                plsc.addupdate_scatter(
                    hist, [dst_v[i, pl.ds(j * 16, 16)]], ones)

        pltpu.sync_copy(hist, out_hbm.at[pl.ds((ch * NW + wid) * NP, NP)])


@jax.jit
def _deg(dst_all):
    return pl.kernel(
        _deg_body,
        out_type=jax.ShapeDtypeStruct((C * NW * NP,), jnp.float32),
        mesh=_mesh(),
        compiler_params=pltpu.CompilerParams(needs_layout_passes=False),
        scratch_types=[
            pltpu.VMEM((NCHUNK, K), jnp.int32),
            pltpu.VMEM((NP,), jnp.float32),
        ],
    )(dst_all)


_BM = 1024


def _dred_body(d_ref, o_ref):
    deg = jnp.sum(d_ref[0], axis=0)
    inv = 1.0 / jnp.maximum(deg, 1.0)
    o_ref[0] = jnp.broadcast_to(inv[:, None], (_BM, DEG_W))


@jax.jit
def _dred(degs):
    return pl.pallas_call(
        _dred_body,
        grid=(C, NP // _BM),
        in_specs=[pl.BlockSpec((1, NW, _BM), lambda ch, i: (ch, 0, i))],
        out_specs=pl.BlockSpec((1, _BM, DEG_W), lambda ch, i: (ch, i, 0)),
        out_shape=jax.ShapeDtypeStruct((C, NP, DEG_W), jnp.float32),
    )(degs)


def _mm_body(x_ref, w_ref, o_ref):
    o_ref[...] = jnp.dot(x_ref[...], w_ref[...],
                         preferred_element_type=jnp.float32)


@jax.jit
def _mm(x, w):
    return pl.pallas_call(
        _mm_body,
        grid=(NP // _BM,),
        in_specs=[pl.BlockSpec((_BM, D), lambda i: (i, 0)),
                  pl.BlockSpec((D, D), lambda i: (0, 0))],
        out_specs=pl.BlockSpec((_BM, D), lambda i: (i, 0)),
        out_shape=jax.ShapeDtypeStruct((NP, D), jnp.float32),
    )(x, w)


def _mmf_body(p0_ref, p1_ref, dq_ref, w_ref, o_ref):
    inv = dq_ref[0, :, 0:1]
    h = jnp.maximum((p0_ref[0] + p1_ref[0]) * inv, 0.0)
    o_ref[...] = jnp.dot(h, w_ref[...], preferred_element_type=jnp.float32)


def _make_mmf(ch):
    def call(p, degq, w):
        return pl.pallas_call(
            _mmf_body,
            grid=(NP // _BM,),
            in_specs=[
                pl.BlockSpec((1, _BM, D), lambda i: (0, i, 0)),
                pl.BlockSpec((1, _BM, D), lambda i: (1, i, 0)),
                pl.BlockSpec((1, _BM, DEG_W), lambda i: (ch, i, 0)),
                pl.BlockSpec((D, D), lambda i: (0, 0)),
            ],
            out_specs=pl.BlockSpec((_BM, D), lambda i: (i, 0)),
            out_shape=jax.ShapeDtypeStruct((NP, D), jnp.float32),
        )(p, p, degq, w)
    return jax.jit(call)


_mmf_ch = [_make_mmf(ch) for ch in range(C)]


def _att_body(q0a, q0b, q1a, q1b, q2a, q2b, dq0, dq1, dq2, att_ref, o_ref):
    hs = []
    ss = []
    for idx, (qa, qb, dq) in enumerate(
            ((q0a, q0b, dq0), (q1a, q1b, dq1), (q2a, q2b, dq2))):
        h = jnp.maximum((qa[0] + qb[0]) * dq[0, :, 0:1], 0.0)
        hs.append(h)
        ss.append(jnp.sum(h * att_ref[idx:idx + 1, :], axis=-1,
                          keepdims=True))
    m = jnp.maximum(jnp.maximum(ss[0], ss[1]), ss[2])
    es = [jnp.exp(sc - m) for sc in ss]
    denom = es[0] + es[1] + es[2]
    o_ref[...] = (es[0] * hs[0] + es[1] * hs[1] + es[2] * hs[2]) / denom


@jax.jit
def _att(q0, q1, q2, degq, att2):
    qspec = lambda part: pl.BlockSpec(
        (1, _BM, D), lambda i, part=part: (part, i, 0))
    dspec = lambda ch: pl.BlockSpec(
        (1, _BM, DEG_W), lambda i, ch=ch: (ch, i, 0))
    return pl.pallas_call(
        _att_body,
        grid=(NP // _BM,),
        in_specs=[qspec(0), qspec(1), qspec(0), qspec(1), qspec(0), qspec(1),
                  dspec(0), dspec(1), dspec(2),
                  pl.BlockSpec((C, D), lambda i: (0, 0))],
        out_specs=pl.BlockSpec((_BM, D), lambda i: (i, 0)),
        out_shape=jax.ShapeDtypeStruct((N, D), jnp.float32),
    )(q0, q0, q1, q1, q2, q2, degq, degq, degq, att2)


def kernel(x, edge_index_0, edge_index_1, edge_index_2,
           W0_0, W0_1, W1_0, W1_1, W2_0, W2_1, att):
    eis = (edge_index_0, edge_index_1, edge_index_2)
    srcs = [ei[0].reshape(NW, NCHUNK, K) for ei in eis]
    dsts = [ei[1].reshape(NW, NCHUNK, K) for ei in eis]
    dst_all = jnp.stack(dsts)

    xp = jnp.zeros((NP, D), jnp.float32).at[:N].set(x)
    zero_agg = jnp.zeros((NP, D), jnp.float32)

    degs = _deg(dst_all).reshape(C, NW, NP)
    degq = _dred(degs)

    qs = []
    for ch, (Wa, Wb) in enumerate(((W0_0, W0_1), (W1_0, W1_1),
                                   (W2_0, W2_1))):
        t0 = _mm(xp, Wa)
        p = _agg(t0, srcs[ch], dsts[ch], zero_agg)
        t1 = _mmf_ch[ch](p, degq, Wb)
        q = _agg(t1, srcs[ch], dsts[ch], zero_agg)
        qs.append(q)

    return _att(qs[0], qs[1], qs[2], degq, att[0])


# speedup vs baseline: 8.2839x; 1.6222x over previous
import functools

import jax
import jax.numpy as jnp
from jax import lax
from jax.experimental import pallas as pl
from jax.experimental.pallas import tpu as pltpu
from jax.experimental.pallas import tpu_sc as plsc

N = 10000
NP = 10240
D = 128
E = 320000
C = 3

NC = 2
NS = 16
NW = NC * NS
EPW = E // NW
K = 80
NCHUNK = EPW // K
RPT = NP // NS
DEG_W = 16

_mesh = functools.partial(
    plsc.VectorSubcoreMesh, core_axis_name="c", subcore_axis_name="s",
    num_cores=NC, num_subcores=NS)


G = 5
NGRP = NCHUNK // G


def _agg_body(t_hbm, src_hbm, dst_hbm, zero_hbm, out_hbm,
              src_v, dst_v, rows_v, acc_sp, gsem, ssem0, ssem1):
    c = lax.axis_index("c")
    s = lax.axis_index("s")
    wid = c * NS + s
    pltpu.sync_copy(src_hbm.at[wid], src_v)
    pltpu.sync_copy(dst_hbm.at[wid], dst_v)
    pltpu.sync_copy(zero_hbm.at[pl.ds(s * RPT, RPT)],
                    acc_sp.at[pl.ds(s * RPT, RPT)])
    plsc.subcore_barrier()

    def gath(ci, buf):
        return pltpu.async_copy(t_hbm.at[src_v.at[pl.ds(ci * K, K)]],
                                rows_v.at[buf], gsem)

    def gath_wait(ci, buf):
        pltpu.make_async_copy(t_hbm.at[src_v.at[pl.ds(ci * K, K)]],
                              rows_v.at[buf], gsem).wait()

    def scat(ci, buf, sem):
        return pltpu.async_copy(rows_v.at[buf], acc_sp.at[dst_v.at[ci]],
                                sem, add=True)

    def scat_wait(ci, buf, sem):
        pltpu.make_async_copy(rows_v.at[buf], acc_sp.at[dst_v.at[ci]],
                              sem).wait()

    gath(0, 0)

    @pl.loop(0, (NCHUNK - 1) // 2)
    def _pair(j):
        c = 2 * j

        @pl.when(j >= 1)
        def _():
            scat_wait(c - 1, 1, ssem1)

        gath(c + 1, 1)
        gath_wait(c, 0)
        scat(c, 0, ssem0)
        scat_wait(c, 0, ssem0)
        gath(c + 2, 0)
        gath_wait(c + 1, 1)
        scat(c + 1, 1, ssem1)

    last = NCHUNK - 1
    scat_wait(last - 1, 1, ssem1)
    gath_wait(last, 0)
    scat(last, 0, ssem0)
    scat_wait(last, 0, ssem0)

    plsc.subcore_barrier()
    pltpu.sync_copy(acc_sp.at[pl.ds(s * RPT, RPT)],
                    out_hbm.at[c, pl.ds(s * RPT, RPT)])


@jax.jit
def _agg(t, src3, dst3, zero_agg):
    return pl.kernel(
        _agg_body,
        out_type=jax.ShapeDtypeStruct((NC, NP, D), jnp.float32),
        mesh=_mesh(),
        scratch_types=[
            pltpu.VMEM((EPW,), jnp.int32),
            pltpu.VMEM((NCHUNK, K), jnp.int32),
            pltpu.VMEM((2, K, D), jnp.float32),
            pltpu.VMEM_SHARED((NP, D), jnp.float32),
            pltpu.SemaphoreType.DMA,
            pltpu.SemaphoreType.DMA,
            pltpu.SemaphoreType.DMA,
        ],
    )(t, src3, dst3, zero_agg)


def _deg_body(dst_hbm, out_hbm, dst_v, hist):
    c = lax.axis_index("c")
    s = lax.axis_index("s")
    wid = c * NS + s
    ones = jnp.ones((16,), jnp.float32)
    for ch in range(C):
        pltpu.sync_copy(dst_hbm.at[ch, wid], dst_v)

        @pl.loop(0, NP // 16)
        def _z(i):
            hist[pl.ds(i * 16, 16)] = jnp.zeros((16,), jnp.float32)

        @pl.loop(0, NCHUNK)
        def _acc(i):
            for j in range(K // 16):
                plsc.addupdate_scatter(
                    hist, [dst_v[i, pl.ds(j * 16, 16)]], ones)

        pltpu.sync_copy(hist, out_hbm.at[pl.ds((ch * NW + wid) * NP, NP)])


@jax.jit
def _deg(dst_all):
    return pl.kernel(
        _deg_body,
        out_type=jax.ShapeDtypeStruct((C * NW * NP,), jnp.float32),
        mesh=_mesh(),
        compiler_params=pltpu.CompilerParams(needs_layout_passes=False),
        scratch_types=[
            pltpu.VMEM((NCHUNK, K), jnp.int32),
            pltpu.VMEM((NP,), jnp.float32),
        ],
    )(dst_all)


_BM = 1024


def _dred_body(d_ref, o_ref):
    deg = jnp.sum(d_ref[0], axis=0)
    inv = 1.0 / jnp.maximum(deg, 1.0)
    o_ref[0] = jnp.broadcast_to(inv[:, None], (_BM, DEG_W))


@jax.jit
def _dred(degs):
    return pl.pallas_call(
        _dred_body,
        grid=(C, NP // _BM),
        in_specs=[pl.BlockSpec((1, NW, _BM), lambda ch, i: (ch, 0, i))],
        out_specs=pl.BlockSpec((1, _BM, DEG_W), lambda ch, i: (ch, i, 0)),
        out_shape=jax.ShapeDtypeStruct((C, NP, DEG_W), jnp.float32),
    )(degs)


def _mm_body(x_ref, w_ref, o_ref):
    o_ref[...] = jnp.dot(x_ref[...], w_ref[...],
                         preferred_element_type=jnp.float32)


@jax.jit
def _mm(x, w):
    return pl.pallas_call(
        _mm_body,
        grid=(NP // _BM,),
        in_specs=[pl.BlockSpec((_BM, D), lambda i: (i, 0)),
                  pl.BlockSpec((D, D), lambda i: (0, 0))],
        out_specs=pl.BlockSpec((_BM, D), lambda i: (i, 0)),
        out_shape=jax.ShapeDtypeStruct((NP, D), jnp.float32),
    )(x, w)


def _mmf_body(p0_ref, p1_ref, dq_ref, w_ref, o_ref):
    inv = dq_ref[0, :, 0:1]
    h = jnp.maximum((p0_ref[0] + p1_ref[0]) * inv, 0.0)
    o_ref[...] = jnp.dot(h, w_ref[...], preferred_element_type=jnp.float32)


def _make_mmf(ch):
    def call(p, degq, w):
        return pl.pallas_call(
            _mmf_body,
            grid=(NP // _BM,),
            in_specs=[
                pl.BlockSpec((1, _BM, D), lambda i: (0, i, 0)),
                pl.BlockSpec((1, _BM, D), lambda i: (1, i, 0)),
                pl.BlockSpec((1, _BM, DEG_W), lambda i: (ch, i, 0)),
                pl.BlockSpec((D, D), lambda i: (0, 0)),
            ],
            out_specs=pl.BlockSpec((_BM, D), lambda i: (i, 0)),
            out_shape=jax.ShapeDtypeStruct((NP, D), jnp.float32),
        )(p, p, degq, w)
    return jax.jit(call)


_mmf_ch = [_make_mmf(ch) for ch in range(C)]


def _att_body(q0a, q0b, q1a, q1b, q2a, q2b, dq0, dq1, dq2, att_ref, o_ref):
    hs = []
    ss = []
    for idx, (qa, qb, dq) in enumerate(
            ((q0a, q0b, dq0), (q1a, q1b, dq1), (q2a, q2b, dq2))):
        h = jnp.maximum((qa[0] + qb[0]) * dq[0, :, 0:1], 0.0)
        hs.append(h)
        ss.append(jnp.sum(h * att_ref[idx:idx + 1, :], axis=-1,
                          keepdims=True))
    m = jnp.maximum(jnp.maximum(ss[0], ss[1]), ss[2])
    es = [jnp.exp(sc - m) for sc in ss]
    denom = es[0] + es[1] + es[2]
    o_ref[...] = (es[0] * hs[0] + es[1] * hs[1] + es[2] * hs[2]) / denom


@jax.jit
def _att(q0, q1, q2, degq, att2):
    qspec = lambda part: pl.BlockSpec(
        (1, _BM, D), lambda i, part=part: (part, i, 0))
    dspec = lambda ch: pl.BlockSpec(
        (1, _BM, DEG_W), lambda i, ch=ch: (ch, i, 0))
    return pl.pallas_call(
        _att_body,
        grid=(NP // _BM,),
        in_specs=[qspec(0), qspec(1), qspec(0), qspec(1), qspec(0), qspec(1),
                  dspec(0), dspec(1), dspec(2),
                  pl.BlockSpec((C, D), lambda i: (0, 0))],
        out_specs=pl.BlockSpec((_BM, D), lambda i: (i, 0)),
        out_shape=jax.ShapeDtypeStruct((N, D), jnp.float32),
    )(q0, q0, q1, q1, q2, q2, degq, degq, degq, att2)


def kernel(x, edge_index_0, edge_index_1, edge_index_2,
           W0_0, W0_1, W1_0, W1_1, W2_0, W2_1, att):
    eis = (edge_index_0, edge_index_1, edge_index_2)
    srcs = [ei[0].reshape(NW, EPW) for ei in eis]
    dsts = [ei[1].reshape(NW, NCHUNK, K) for ei in eis]
    dst_all = jnp.stack(dsts)

    xp = jnp.zeros((NP, D), jnp.float32).at[:N].set(x)
    zero_agg = jnp.zeros((NP, D), jnp.float32)

    degs = _deg(dst_all).reshape(C, NW, NP)
    degq = _dred(degs)

    qs = []
    for ch, (Wa, Wb) in enumerate(((W0_0, W0_1), (W1_0, W1_1),
                                   (W2_0, W2_1))):
        t0 = _mm(xp, Wa)
        p = _agg(t0, srcs[ch], dsts[ch], zero_agg)
        t1 = _mmf_ch[ch](p, degq, Wb)
        q = _agg(t1, srcs[ch], dsts[ch], zero_agg)
        qs.append(q)

    return _att(qs[0], qs[1], qs[2], degq, att[0])
